# baseline (speedup 1.0000x reference)
import functools

import jax
import jax.numpy as jnp
from jax import lax
from jax.experimental import pallas as pl
from jax.experimental.pallas import tpu as pltpu
from jax.experimental.pallas import tpu_sc as plsc

_MARGIN = 2.0
_B, _H, _W, _D = 4, 224, 224, 96
_DP = 128
_M = 16
_LPAD = 32
_N = _H * _W
_PIX = _B * _N
_P = 7168
_HB = 16
_NW = 32
_PPW = _PIX // _NW
_NHALF = 2
_PIX2 = _PIX // _NHALF
_PPW2 = _PIX2 // _NW
_CH = 112
_NCHUNK = _PPW2 // _CH


def _tpad_body(iref, oref):
    x = iref[...]
    for i in range(_HB):
        xt = jnp.transpose(x[i])
        oref[pl.ds(i * _W, _W), :] = jnp.concatenate(
            [xt, jnp.zeros((_W, _DP - _D), jnp.float32)], axis=1)


def _transpose_pad(emb_t):
    rows = emb_t.shape[0]
    return pl.pallas_call(
        _tpad_body,
        grid=(rows // _HB,),
        in_specs=[pl.BlockSpec((_HB, _D, _W), lambda i: (i, 0, 0))],
        out_specs=pl.BlockSpec((_HB * _W, _DP), lambda i: (i, 0)),
        out_shape=jax.ShapeDtypeStruct((rows * _W, _DP), jnp.float32),
    )(emb_t)


def _labels_counts_body(mref, lref, cref):
    b = pl.program_id(0)
    p = pl.program_id(1)
    m = mref[0]
    w = lax.broadcasted_iota(jnp.int32, (_M, _P), 0) + 1
    lbl = jnp.max(jnp.where(m > 0.5, w, 0), axis=0, keepdims=True)
    lref[...] = lbl.reshape(_P)
    oh = jnp.broadcast_to(lbl, (_LPAD, _P)) == lax.broadcasted_iota(
        jnp.int32, (_LPAD, _P), 0)
    cnt = jnp.sum(oh.astype(jnp.float32), axis=1, keepdims=True)

    @pl.when((p == 0) & (b == 0))
    def _():
        cref[...] = jnp.zeros_like(cref)

    col = lax.broadcasted_iota(jnp.int32, (_LPAD, _B), 1) == b
    cref[...] += jnp.where(col, cnt, 0.0)


def _labels_counts(masks3):
    return pl.pallas_call(
        _labels_counts_body,
        grid=(_B, _N // _P),
        in_specs=[pl.BlockSpec((1, _M, _P), lambda b, p: (b, 0, p))],
        out_specs=[
            pl.BlockSpec((_P,), lambda b, p: (b * (_N // _P) + p,)),
            pl.BlockSpec((_LPAD, _B), lambda b, p: (0, 0)),
        ],
        out_shape=[
            jax.ShapeDtypeStruct((_PIX,), jnp.int32),
            jax.ShapeDtypeStruct((_LPAD, _B), jnp.float32),
        ],
    )(masks3)


def _segment_sums(emb_half, labels_flat, half):
    mesh = plsc.VectorSubcoreMesh(core_axis_name="c", subcore_axis_name="s")

    @functools.partial(
        pl.kernel,
        mesh=mesh,
        out_type=jax.ShapeDtypeStruct((_NW, _LPAD, _DP), jnp.float32),
        scratch_types=[
            pltpu.VMEM((_CH, _DP), jnp.float32),
            pltpu.VMEM((_CH, _DP), jnp.float32),
            pltpu.VMEM((_CH,), jnp.int32),
            pltpu.VMEM((_CH,), jnp.int32),
            pltpu.VMEM((_CH,), jnp.int32),
            pltpu.VMEM((_CH,), jnp.int32),
            pltpu.VMEM((_LPAD, _DP), jnp.float32),
            pltpu.VMEM_SHARED((16 * _LPAD, _DP), jnp.float32),
            pltpu.SemaphoreType.DMA,
            pltpu.SemaphoreType.DMA,
            pltpu.SemaphoreType.DMA,
            pltpu.SemaphoreType.DMA,
        ],
    )
    def seg(emb_hbm, lab_hbm, out_hbm, ebuf0, ebuf1, lbuf0, lbuf1,
            ibuf0, ibuf1, zbuf, acc, se0, se1, sl0, sl1):
        c = lax.axis_index("c")
        s = lax.axis_index("s")
        wid = c * 16 + s
        base = wid * _PPW2
        lbase = half * _PIX2 + base
        bufs = ((ebuf0, lbuf0, ibuf0, se0, sl0),
                (ebuf1, lbuf1, ibuf1, se1, sl1))

        def load_copies(j):
            eb, lb, _, se, sl = bufs[j % 2]
            off = base + j * _CH
            loff = lbase + j * _CH
            return (pltpu.make_async_copy(emb_hbm.at[pl.ds(off, _CH)], eb, se),
                    pltpu.make_async_copy(lab_hbm.at[pl.ds(loff, _CH)], lb, sl))

        for r in range(_LPAD):
            for k in range(_DP // 16):
                zbuf[r, pl.ds(k * 16, 16)] = jnp.zeros((16,), jnp.float32)
        pltpu.sync_copy(zbuf, acc.at[pl.ds(s * _LPAD, _LPAD)])

        for cp in load_copies(0):
            cp.start()
        for j in range(_NCHUNK):
            for cp in load_copies(j):
                cp.wait()
            if j + 1 < _NCHUNK:
                for cp in load_copies(j + 1):
                    cp.start()
            eb, lb, ib, _, _ = bufs[j % 2]
            for k in range(_CH // 16):
                ib[pl.ds(k * 16, 16)] = lb[pl.ds(k * 16, 16)] + s * _LPAD
            pltpu.sync_copy(eb, acc.at[ib], add=True)

        pltpu.sync_copy(acc.at[pl.ds(s * _LPAD, _LPAD)], out_hbm.at[wid])

    return seg(emb_half, labels_flat)


def _epilogue_body(paref, pbref, cref, oref):
    sums = jnp.concatenate(
        [jnp.sum(paref[...], axis=1), jnp.sum(pbref[...], axis=1)], axis=0)
    cnts = cref[...]
    total = jnp.zeros((), jnp.float32)
    vb = jnp.zeros((), jnp.float32)
    for b in range(_B):
        sb = sums[b]
        cnt = cnts[:, b:b + 1]
        present = cnt > 0.0
        cent = jnp.where(present, sb / jnp.maximum(cnt, 1.0), 0.0)
        nrm = jnp.sum(cent * cent, axis=1, keepdims=True)
        g = lax.dot_general(cent, cent, (((1,), (1,)), ((), ())),
                            preferred_element_type=jnp.float32)
        d2 = jnp.maximum(nrm + jnp.transpose(nrm) - 2.0 * g, 0.0)
        dist = jnp.sqrt(d2)
        ii = lax.broadcasted_iota(jnp.int32, (_LPAD, _LPAD), 0)
        jj = lax.broadcasted_iota(jnp.int32, (_LPAD, _LPAD), 1)
        vp = present & jnp.transpose(present) & (ii < jj)
        hinge = jnp.where(vp, jnp.maximum(_MARGIN - dist, 0.0), 0.0)
        pair_loss = jnp.sum(hinge)
        n = jnp.sum(present.astype(jnp.float32))
        npair = n * (n - 1.0) * 0.5
        valid = n >= 2.0
        total = total + jnp.where(valid, pair_loss / jnp.maximum(npair, 1.0),
                                  0.0)
        vb = vb + jnp.where(valid, 1.0, 0.0)
    out = jnp.where(vb == 0.0, 0.0, total / jnp.maximum(vb, 1.0))
    oref[...] = jnp.broadcast_to(out, (1, 1))


def _epilogue(partials_a, partials_b, counts):
    return pl.pallas_call(
        _epilogue_body,
        out_shape=jax.ShapeDtypeStruct((1, 1), jnp.float32),
    )(partials_a, partials_b, counts)


def kernel(embeddings, masks):
    masks3 = masks.reshape(_B, _M, _N)
    labels, counts = _labels_counts(masks3)
    emb_t = jnp.transpose(embeddings, (0, 1, 3, 2)).reshape(_B * _H, _D, _W)
    rows_half = _B * _H // _NHALF
    pad_a = _transpose_pad(emb_t[:rows_half])
    part_a = _segment_sums(pad_a, labels, 0)
    pad_b = _transpose_pad(emb_t[rows_half:])
    part_b = _segment_sums(pad_b, labels, 1)
    loss = _epilogue(part_a.reshape(2, _NW // 2, _LPAD, _DP),
                     part_b.reshape(2, _NW // 2, _LPAD, _DP), counts)
    return loss[0, 0]


# speedup vs baseline: 1.2549x; 1.2549x over previous
import functools

import jax
import jax.numpy as jnp
from jax import lax
from jax.experimental import pallas as pl
from jax.experimental.pallas import tpu as pltpu
from jax.experimental.pallas import tpu_sc as plsc

_MARGIN = 2.0
_B, _H, _W, _D = 4, 224, 224, 96
_DP = 128
_M = 16
_LPAD = 32
_N = _H * _W
_PIX = _B * _N
_P = 7168
_HB = 32
_NW = 32
_PPW = _PIX // _NW
_CH = 128
_NCHUNK = _PPW // _CH


def _tpad_body(iref, oref):
    x = iref[...]
    for i in range(_HB):
        xt = jnp.transpose(x[i])
        oref[pl.ds(i * _W, _W), :] = jnp.concatenate(
            [xt, jnp.zeros((_W, _DP - _D), jnp.float32)], axis=1)


def _transpose_pad(emb_t):
    return pl.pallas_call(
        _tpad_body,
        grid=(_B * _H // _HB,),
        in_specs=[pl.BlockSpec((_HB, _D, _W), lambda i: (i, 0, 0))],
        out_specs=pl.BlockSpec((_HB * _W, _DP), lambda i: (i, 0)),
        out_shape=jax.ShapeDtypeStruct((_PIX, _DP), jnp.float32),
    )(emb_t)


def _labels_counts_body(mref, lref, cref):
    b = pl.program_id(0)
    p = pl.program_id(1)
    m = mref[0]
    w = lax.broadcasted_iota(jnp.int32, (_M, _P), 0) + 1
    lbl = jnp.max(jnp.where(m > 0.5, w, 0), axis=0, keepdims=True)
    lref[...] = lbl.reshape(_P)
    oh = jnp.broadcast_to(lbl, (_LPAD, _P)) == lax.broadcasted_iota(
        jnp.int32, (_LPAD, _P), 0)
    cnt = jnp.sum(oh.astype(jnp.float32), axis=1, keepdims=True)

    @pl.when((p == 0) & (b == 0))
    def _():
        cref[...] = jnp.zeros_like(cref)

    col = lax.broadcasted_iota(jnp.int32, (_LPAD, _B), 1) == b
    cref[...] += jnp.where(col, cnt, 0.0)


def _labels_counts(masks3):
    return pl.pallas_call(
        _labels_counts_body,
        grid=(_B, _N // _P),
        in_specs=[pl.BlockSpec((1, _M, _P), lambda b, p: (b, 0, p))],
        out_specs=[
            pl.BlockSpec((_P,), lambda b, p: (b * (_N // _P) + p,)),
            pl.BlockSpec((_LPAD, _B), lambda b, p: (0, 0)),
        ],
        out_shape=[
            jax.ShapeDtypeStruct((_PIX,), jnp.int32),
            jax.ShapeDtypeStruct((_LPAD, _B), jnp.float32),
        ],
    )(masks3)


def _segment_sums(emb_flat, labels_flat):
    mesh = plsc.VectorSubcoreMesh(core_axis_name="c", subcore_axis_name="s")

    @functools.partial(
        pl.kernel,
        mesh=mesh,
        out_type=jax.ShapeDtypeStruct((_NW, _LPAD, _DP), jnp.float32),
        scratch_types=[
            pltpu.VMEM((_CH, _DP), jnp.float32),
            pltpu.VMEM((_CH, _DP), jnp.float32),
            pltpu.VMEM((_CH,), jnp.int32),
            pltpu.VMEM((_CH,), jnp.int32),
            pltpu.VMEM((_CH,), jnp.int32),
            pltpu.VMEM((_CH,), jnp.int32),
            pltpu.VMEM((_LPAD, _DP), jnp.float32),
            pltpu.VMEM_SHARED((16 * _LPAD, _DP), jnp.float32),
            pltpu.SemaphoreType.DMA,
            pltpu.SemaphoreType.DMA,
            pltpu.SemaphoreType.DMA,
            pltpu.SemaphoreType.DMA,
        ],
    )
    def seg(emb_hbm, lab_hbm, out_hbm, ebuf0, ebuf1, lbuf0, lbuf1,
            ibuf0, ibuf1, zbuf, acc, se0, se1, sl0, sl1):
        c = lax.axis_index("c")
        s = lax.axis_index("s")
        wid = c * 16 + s
        base = wid * _PPW
        bufs = ((ebuf0, lbuf0, ibuf0, se0, sl0),
                (ebuf1, lbuf1, ibuf1, se1, sl1))

        def load_copies(j):
            eb, lb, _, se, sl = bufs[j % 2]
            off = base + j * _CH
            return (pltpu.make_async_copy(emb_hbm.at[pl.ds(off, _CH)], eb, se),
                    pltpu.make_async_copy(lab_hbm.at[pl.ds(off, _CH)], lb, sl))

        for r in range(_LPAD):
            for k in range(_DP // 16):
                zbuf[r, pl.ds(k * 16, 16)] = jnp.zeros((16,), jnp.float32)
        pltpu.sync_copy(zbuf, acc.at[pl.ds(s * _LPAD, _LPAD)])

        for cp in load_copies(0):
            cp.start()
        for j in range(_NCHUNK):
            for cp in load_copies(j):
                cp.wait()
            if j + 1 < _NCHUNK:
                for cp in load_copies(j + 1):
                    cp.start()
            eb, lb, ib, _, _ = bufs[j % 2]
            for k in range(_CH // 16):
                ib[pl.ds(k * 16, 16)] = lb[pl.ds(k * 16, 16)] + s * _LPAD
            pltpu.sync_copy(eb, acc.at[ib], add=True)

        pltpu.sync_copy(acc.at[pl.ds(s * _LPAD, _LPAD)], out_hbm.at[wid])

    return seg(emb_flat, labels_flat)


def _epilogue_body(pref, cref, oref):
    sums = jnp.sum(pref[...], axis=1)
    cnts = cref[...]
    total = jnp.zeros((), jnp.float32)
    vb = jnp.zeros((), jnp.float32)
    for b in range(_B):
        sb = sums[b]
        cnt = cnts[:, b:b + 1]
        present = cnt > 0.0
        cent = jnp.where(present, sb / jnp.maximum(cnt, 1.0), 0.0)
        nrm = jnp.sum(cent * cent, axis=1, keepdims=True)
        g = lax.dot_general(cent, cent, (((1,), (1,)), ((), ())),
                            preferred_element_type=jnp.float32)
        d2 = jnp.maximum(nrm + jnp.transpose(nrm) - 2.0 * g, 0.0)
        dist = jnp.sqrt(d2)
        ii = lax.broadcasted_iota(jnp.int32, (_LPAD, _LPAD), 0)
        jj = lax.broadcasted_iota(jnp.int32, (_LPAD, _LPAD), 1)
        vp = present & jnp.transpose(present) & (ii < jj)
        hinge = jnp.where(vp, jnp.maximum(_MARGIN - dist, 0.0), 0.0)
        pair_loss = jnp.sum(hinge)
        n = jnp.sum(present.astype(jnp.float32))
        npair = n * (n - 1.0) * 0.5
        valid = n >= 2.0
        total = total + jnp.where(valid, pair_loss / jnp.maximum(npair, 1.0),
                                  0.0)
        vb = vb + jnp.where(valid, 1.0, 0.0)
    out = jnp.where(vb == 0.0, 0.0, total / jnp.maximum(vb, 1.0))
    oref[...] = jnp.broadcast_to(out, (1, 1))


def _epilogue(partials, counts):
    return pl.pallas_call(
        _epilogue_body,
        out_shape=jax.ShapeDtypeStruct((1, 1), jnp.float32),
    )(partials, counts)


def kernel(embeddings, masks):
    masks3 = masks.reshape(_B, _M, _N)
    labels, counts = _labels_counts(masks3)
    emb_t = jnp.transpose(embeddings, (0, 1, 3, 2)).reshape(_B * _H, _D, _W)
    emb_pad = _transpose_pad(emb_t)
    partials = _segment_sums(emb_pad, labels)
    loss = _epilogue(partials.reshape(_B, _NW // _B, _LPAD, _DP), counts)
    return loss[0, 0]


# speedup vs baseline: 1.2796x; 1.0197x over previous
import functools

import jax
import jax.numpy as jnp
from jax import lax
from jax.experimental import pallas as pl
from jax.experimental.pallas import tpu as pltpu
from jax.experimental.pallas import tpu_sc as plsc

_MARGIN = 2.0
_B, _H, _W, _D = 4, 224, 224, 96
_DP = 128
_M = 16
_LPAD = 32
_N = _H * _W
_PIX = _B * _N
_P = 7168
_HB = 64
_NW = 32
_PPW = _PIX // _NW
_CH = 128
_NCHUNK = _PPW // _CH


def _tpad_body(iref, oref):
    x = iref[...]
    for i in range(_HB):
        xt = jnp.transpose(x[i])
        oref[pl.ds(i * _W, _W), :] = jnp.concatenate(
            [xt, jnp.zeros((_W, _DP - _D), jnp.float32)], axis=1)


def _transpose_pad(emb_t):
    return pl.pallas_call(
        _tpad_body,
        grid=(_B * _H // _HB,),
        in_specs=[pl.BlockSpec((_HB, _D, _W), lambda i: (i, 0, 0))],
        out_specs=pl.BlockSpec((_HB * _W, _DP), lambda i: (i, 0)),
        out_shape=jax.ShapeDtypeStruct((_PIX, _DP), jnp.float32),
    )(emb_t)


def _labels_counts_body(mref, lref, cref):
    b = pl.program_id(0)
    p = pl.program_id(1)
    m = mref[0]
    w = lax.broadcasted_iota(jnp.int32, (_M, _P), 0) + 1
    lbl = jnp.max(jnp.where(m > 0.5, w, 0), axis=0, keepdims=True)
    lref[...] = lbl.reshape(_P)
    oh = jnp.broadcast_to(lbl, (_LPAD, _P)) == lax.broadcasted_iota(
        jnp.int32, (_LPAD, _P), 0)
    cnt = jnp.sum(oh.astype(jnp.float32), axis=1, keepdims=True)

    @pl.when((p == 0) & (b == 0))
    def _():
        cref[...] = jnp.zeros_like(cref)

    col = lax.broadcasted_iota(jnp.int32, (_LPAD, _B), 1) == b
    cref[...] += jnp.where(col, cnt, 0.0)


def _labels_counts(masks3):
    return pl.pallas_call(
        _labels_counts_body,
        grid=(_B, _N // _P),
        in_specs=[pl.BlockSpec((1, _M, _P), lambda b, p: (b, 0, p))],
        out_specs=[
            pl.BlockSpec((_P,), lambda b, p: (b * (_N // _P) + p,)),
            pl.BlockSpec((_LPAD, _B), lambda b, p: (0, 0)),
        ],
        out_shape=[
            jax.ShapeDtypeStruct((_PIX,), jnp.int32),
            jax.ShapeDtypeStruct((_LPAD, _B), jnp.float32),
        ],
    )(masks3)


def _segment_sums(emb_flat, labels_flat):
    mesh = plsc.VectorSubcoreMesh(core_axis_name="c", subcore_axis_name="s")

    @functools.partial(
        pl.kernel,
        mesh=mesh,
        out_type=jax.ShapeDtypeStruct((_NW, _LPAD, _DP), jnp.float32),
        scratch_types=[
            pltpu.VMEM((_CH, _DP), jnp.float32),
            pltpu.VMEM((_CH, _DP), jnp.float32),
            pltpu.VMEM((_CH,), jnp.int32),
            pltpu.VMEM((_CH,), jnp.int32),
            pltpu.VMEM((_CH,), jnp.int32),
            pltpu.VMEM((_CH,), jnp.int32),
            pltpu.VMEM((_LPAD, _DP), jnp.float32),
            pltpu.VMEM_SHARED((16 * _LPAD, _DP), jnp.float32),
            pltpu.SemaphoreType.DMA,
            pltpu.SemaphoreType.DMA,
            pltpu.SemaphoreType.DMA,
            pltpu.SemaphoreType.DMA,
        ],
    )
    def seg(emb_hbm, lab_hbm, out_hbm, ebuf0, ebuf1, lbuf0, lbuf1,
            ibuf0, ibuf1, zbuf, acc, se0, se1, sl0, sl1):
        c = lax.axis_index("c")
        s = lax.axis_index("s")
        wid = c * 16 + s
        base = wid * _PPW
        bufs = ((ebuf0, lbuf0, ibuf0, se0, sl0),
                (ebuf1, lbuf1, ibuf1, se1, sl1))

        def load_copies(j):
            eb, lb, _, se, sl = bufs[j % 2]
            off = base + j * _CH
            return (pltpu.make_async_copy(emb_hbm.at[pl.ds(off, _CH)], eb, se),
                    pltpu.make_async_copy(lab_hbm.at[pl.ds(off, _CH)], lb, sl))

        for r in range(_LPAD):
            for k in range(_DP // 16):
                zbuf[r, pl.ds(k * 16, 16)] = jnp.zeros((16,), jnp.float32)
        pltpu.sync_copy(zbuf, acc.at[pl.ds(s * _LPAD, _LPAD)])

        for cp in load_copies(0):
            cp.start()
        for j in range(_NCHUNK):
            for cp in load_copies(j):
                cp.wait()
            if j + 1 < _NCHUNK:
                for cp in load_copies(j + 1):
                    cp.start()
            eb, lb, ib, _, _ = bufs[j % 2]
            for k in range(_CH // 16):
                ib[pl.ds(k * 16, 16)] = lb[pl.ds(k * 16, 16)] + s * _LPAD
            pltpu.sync_copy(eb, acc.at[ib], add=True)

        pltpu.sync_copy(acc.at[pl.ds(s * _LPAD, _LPAD)], out_hbm.at[wid])

    return seg(emb_flat, labels_flat)


def _epilogue_body(pref, cref, oref):
    sums = jnp.sum(pref[...], axis=1)
    cnts = cref[...]
    total = jnp.zeros((), jnp.float32)
    vb = jnp.zeros((), jnp.float32)
    for b in range(_B):
        sb = sums[b]
        cnt = cnts[:, b:b + 1]
        present = cnt > 0.0
        cent = jnp.where(present, sb / jnp.maximum(cnt, 1.0), 0.0)
        nrm = jnp.sum(cent * cent, axis=1, keepdims=True)
        g = lax.dot_general(cent, cent, (((1,), (1,)), ((), ())),
                            preferred_element_type=jnp.float32)
        d2 = jnp.maximum(nrm + jnp.transpose(nrm) - 2.0 * g, 0.0)
        dist = jnp.sqrt(d2)
        ii = lax.broadcasted_iota(jnp.int32, (_LPAD, _LPAD), 0)
        jj = lax.broadcasted_iota(jnp.int32, (_LPAD, _LPAD), 1)
        vp = present & jnp.transpose(present) & (ii < jj)
        hinge = jnp.where(vp, jnp.maximum(_MARGIN - dist, 0.0), 0.0)
        pair_loss = jnp.sum(hinge)
        n = jnp.sum(present.astype(jnp.float32))
        npair = n * (n - 1.0) * 0.5
        valid = n >= 2.0
        total = total + jnp.where(valid, pair_loss / jnp.maximum(npair, 1.0),
                                  0.0)
        vb = vb + jnp.where(valid, 1.0, 0.0)
    out = jnp.where(vb == 0.0, 0.0, total / jnp.maximum(vb, 1.0))
    oref[...] = jnp.broadcast_to(out, (1, 1))


def _epilogue(partials, counts):
    return pl.pallas_call(
        _epilogue_body,
        out_shape=jax.ShapeDtypeStruct((1, 1), jnp.float32),
    )(partials, counts)


def kernel(embeddings, masks):
    masks3 = masks.reshape(_B, _M, _N)
    labels, counts = _labels_counts(masks3)
    emb_t = jnp.transpose(embeddings, (0, 1, 3, 2)).reshape(_B * _H, _D, _W)
    emb_pad = _transpose_pad(emb_t)
    partials = _segment_sums(emb_pad, labels)
    loss = _epilogue(partials.reshape(_B, _NW // _B, _LPAD, _DP), counts)
    return loss[0, 0]
